# Initial kernel scaffold; baseline (speedup 1.0000x reference)
import functools

import jax
import jax.numpy as jnp
from jax import lax
from jax.experimental import pallas as pl
from jax.experimental.pallas import tpu as pltpu
from jax.experimental.pallas import tpu_sc as plsc

N = 100000
E = 1600000
AF = 39
H = 50
B = 1024
DEPTH = 3

CH = 25600
NPAD = 4 * CH
ACC = CH + 16
KE = 2000
SB = 80
NSB = KE // SB
BE = 4000
BN = 1000


def _make_segsum(interpret=False):
    EPT = E // 16
    NBLK = EPT // KE
    ZPT = CH // 16
    mesh = plsc.VectorSubcoreMesh(core_axis_name="c", subcore_axis_name="s")

    def body(msg_hbm, dst_hbm, zer_hbm, out_hbm, acc, msgb, dstb, idxb):
        c = lax.axis_index("c")
        s = lax.axis_index("s")
        ebase = s * EPT
        rbase = ebase // SB
        dvec = CH + lax.broadcasted_iota(jnp.int32, (16,), 0)
        for half in range(2):
            ch = c + 2 * half
            nbase = ch * CH
            pltpu.sync_copy(zer_hbm.at[pl.ds(s * ZPT, ZPT)],
                            acc.at[pl.ds(s * ZPT, ZPT)])

            @pl.when(s == 0)
            def _():
                pltpu.sync_copy(zer_hbm.at[pl.ds(0, 16)], acc.at[pl.ds(CH, 16)])

            plsc.subcore_barrier()

            @pl.loop(0, NBLK)
            def _blk(b):
                e0 = ebase + b * KE
                pltpu.sync_copy(msg_hbm.at[pl.ds(e0, KE)], msgb)
                pltpu.sync_copy(dst_hbm.at[pl.ds(rbase + b * NSB, NSB)], dstb)

                @pl.loop(0, NSB)
                def _sub(j):
                    for t in range(SB // 16):
                        v = dstb[j, pl.ds(t * 16, 16)]
                        loc = v - nbase
                        ok = (loc >= 0) & (loc < CH)
                        idxb[j, pl.ds(t * 16, 16)] = jnp.where(ok, loc, dvec)
                    pltpu.sync_copy(msgb.at[pl.ds(j * SB, SB)],
                                    acc.at[idxb.at[j]], add=True)

            plsc.subcore_barrier()
            pltpu.sync_copy(acc.at[pl.ds(s * ZPT, ZPT)],
                            out_hbm.at[pl.ds(nbase + s * ZPT, ZPT)])
            plsc.subcore_barrier()

    return pl.kernel(
        body,
        out_type=jax.ShapeDtypeStruct((NPAD, H), jnp.float32),
        mesh=mesh,
        scratch_types=[
            pltpu.VMEM_SHARED((ACC, H), jnp.float32),
            pltpu.VMEM((KE, H), jnp.float32),
            pltpu.VMEM((NSB, SB), jnp.int32),
            pltpu.VMEM((NSB, SB), jnp.int32),
        ],
        interpret=interpret,
    )


def _make_gather(interpret=False):
    EPT = E // 32
    NBLK = EPT // KE
    mesh = plsc.VectorSubcoreMesh(core_axis_name="c", subcore_axis_name="s")

    def body(ns_hbm, src_hbm, out_hbm, rowsb, srcb, sem):
        c = lax.axis_index("c")
        s = lax.axis_index("s")
        ebase = (s * 2 + c) * EPT
        rbase = ebase // SB

        @pl.loop(0, NBLK)
        def _blk(b):
            e0 = ebase + b * KE
            pltpu.sync_copy(src_hbm.at[pl.ds(rbase + b * NSB, NSB)], srcb)
            cps = [pltpu.async_copy(ns_hbm.at[srcb.at[j]],
                                    rowsb.at[pl.ds(j * SB, SB)], sem)
                   for j in range(NSB)]
            for cp in cps:
                cp.wait()
            pltpu.sync_copy(rowsb, out_hbm.at[pl.ds(e0, KE)])

    return pl.kernel(
        body,
        out_type=jax.ShapeDtypeStruct((E, H), jnp.float32),
        mesh=mesh,
        scratch_types=[
            pltpu.VMEM((KE, H), jnp.float32),
            pltpu.VMEM((NSB, SB), jnp.int32),
            pltpu.SemaphoreType.DMA,
        ],
        interpret=interpret,
    )


def _tc_init(fb_p, wi2):
    def body(fb_ref, w_ref, bin_ref, msg_ref):
        t = jnp.dot(fb_ref[...], w_ref[...], preferred_element_type=jnp.float32)
        bin_ref[...] = t
        msg_ref[...] = jnp.maximum(t, 0.0)

    return pl.pallas_call(
        body,
        grid=((E // 2) // BE,),
        in_specs=[
            pl.BlockSpec((BE, 2 * H), lambda i: (i, 0)),
            pl.BlockSpec((2 * H, 2 * H), lambda i: (0, 0)),
        ],
        out_specs=[pl.BlockSpec((BE, 2 * H), lambda i: (i, 0))] * 2,
        out_shape=[jax.ShapeDtypeStruct((E // 2, 2 * H), jnp.float32)] * 2,
    )(fb_p, wi2)


def _tc_round(nei_p, msg_p, bin_p, wh2, wh2s):
    def body(nei_ref, msg_ref, bin_ref, w_ref, ws_ref, out_ref):
        t = jnp.dot(nei_ref[...], w_ref[...], preferred_element_type=jnp.float32)
        t = t - jnp.dot(msg_ref[...], ws_ref[...],
                        preferred_element_type=jnp.float32)
        out_ref[...] = jnp.maximum(bin_ref[...] + t, 0.0)

    return pl.pallas_call(
        body,
        grid=((E // 2) // BE,),
        in_specs=[
            pl.BlockSpec((BE, 2 * H), lambda i: (i, 0)),
            pl.BlockSpec((BE, 2 * H), lambda i: (i, 0)),
            pl.BlockSpec((BE, 2 * H), lambda i: (i, 0)),
            pl.BlockSpec((2 * H, 2 * H), lambda i: (0, 0)),
            pl.BlockSpec((2 * H, 2 * H), lambda i: (0, 0)),
        ],
        out_specs=pl.BlockSpec((BE, 2 * H), lambda i: (i, 0)),
        out_shape=jax.ShapeDtypeStruct((E // 2, 2 * H), jnp.float32),
    )(nei_p, msg_p, bin_p, wh2, wh2s)


def _tc_readout(fatoms, a_msg, mol3, wo1e, wo2e, boe):
    nblk = N // BN

    def body(fa_ref, am_ref, mol_ref, w1_ref, w2_ref, bo_ref, out_ref, acc):
        i = pl.program_id(0)
        h = jnp.dot(fa_ref[...], w1_ref[...], preferred_element_type=jnp.float32)
        h = h + jnp.dot(am_ref[...], w2_ref[...],
                        preferred_element_type=jnp.float32)
        h = jnp.maximum(h + bo_ref[...], 0.0)
        mol = mol_ref[0]
        oh = (mol == lax.broadcasted_iota(jnp.int32, (B, BN), 0)
              ).astype(jnp.float32)
        contrib = jnp.dot(oh, h, preferred_element_type=jnp.float32)

        @pl.when(i == 0)
        def _():
            acc[...] = jnp.zeros_like(acc)

        acc[...] += contrib

        @pl.when(i == nblk - 1)
        def _():
            a = acc[...]
            out_ref[...] = a[:, :H] / jnp.maximum(a[:, H:H + 1], 1.0)

    return pl.pallas_call(
        body,
        grid=(nblk,),
        in_specs=[
            pl.BlockSpec((BN, AF), lambda i: (i, 0)),
            pl.BlockSpec((BN, H), lambda i: (i, 0)),
            pl.BlockSpec((1, 1, BN), lambda i: (i, 0, 0)),
            pl.BlockSpec((AF, H + 1), lambda i: (0, 0)),
            pl.BlockSpec((H, H + 1), lambda i: (0, 0)),
            pl.BlockSpec((1, H + 1), lambda i: (0, 0)),
        ],
        out_specs=pl.BlockSpec((B, H), lambda i: (0, 0)),
        out_shape=jax.ShapeDtypeStruct((B, H), jnp.float32),
        scratch_shapes=[pltpu.VMEM((B, H + 1), jnp.float32)],
    )(fatoms, a_msg, mol3, wo1e, wo2e, boe)


def _block_diag(w):
    z = jnp.zeros_like(w)
    return jnp.concatenate(
        [jnp.concatenate([w, z], 1), jnp.concatenate([z, w], 1)], 0)


def _anti_block_diag(w):
    z = jnp.zeros_like(w)
    return jnp.concatenate(
        [jnp.concatenate([z, w], 1), jnp.concatenate([w, z], 1)], 0)


def kernel(fatoms, fbonds, edge_index, mol_ids, W_i, W_h, W_o, b_o):
    src2 = edge_index[0].reshape(E // SB, SB)
    dst2 = edge_index[1].reshape(E // SB, SB)
    fb_p = fbonds.reshape(E // 2, 2 * H)
    zer = jnp.zeros((ACC, H), jnp.float32)

    wi2 = _block_diag(W_i)
    wh2 = _block_diag(W_h)
    wh2s = _anti_block_diag(W_h)
    zcol = jnp.zeros((W_o.shape[0], 1), jnp.float32)
    wo_e = jnp.concatenate([W_o, zcol], 1)
    wo1e = wo_e[:AF]
    wo2e = wo_e[AF:]
    boe = jnp.concatenate([b_o, jnp.ones((1,), jnp.float32)])[None, :]

    segsum = _make_segsum()
    gather = _make_gather()

    bin_p, msg_p = _tc_init(fb_p, wi2)
    for _ in range(DEPTH - 1):
        ns = segsum(msg_p.reshape(E, H), dst2, zer)
        nei = gather(ns, src2)
        msg_p = _tc_round(nei.reshape(E // 2, 2 * H), msg_p, bin_p, wh2, wh2s)

    a_msg = segsum(msg_p.reshape(E, H), dst2, zer)
    mol3 = mol_ids.reshape(N // BN, 1, BN)
    return _tc_readout(fatoms, a_msg, mol3, wo1e, wo2e, boe)


# speedup vs baseline: 2.8631x; 2.8631x over previous
import functools

import jax
import jax.numpy as jnp
from jax import lax
from jax.experimental import pallas as pl
from jax.experimental.pallas import tpu as pltpu
from jax.experimental.pallas import tpu_sc as plsc

N = 100000
E = 1600000
AF = 39
H = 50
B = 1024
DEPTH = 3

CH = 25600
NPAD = 4 * CH
ACC = CH + 16
KE = 320
SB = 80
NSB = KE // SB
NSLAB = E // KE
HP = 64
BE = 4000
BN = 1000


def _make_segsum(interpret=False):
    NB = -(-NSLAB // 16)
    ZPT = CH // 16
    mesh = plsc.VectorSubcoreMesh(core_axis_name="c", subcore_axis_name="s")

    def body(msg_hbm, dst_hbm, zer_hbm, out_hbm, acc, msgb, dstb, *idxs):
        c = lax.axis_index("c")
        s = lax.axis_index("s")
        dvec = CH + lax.broadcasted_iota(jnp.int32, (16,), 0)
        for half in range(2):
            ch = c + 2 * half
            nbase = ch * CH
            pltpu.sync_copy(zer_hbm.at[pl.ds(s * ZPT, ZPT)],
                            acc.at[pl.ds(s * ZPT, ZPT)])

            @pl.when(s == 0)
            def _():
                pltpu.sync_copy(zer_hbm.at[pl.ds(0, 16)], acc.at[pl.ds(CH, 16)])

            plsc.subcore_barrier()

            @pl.loop(0, NB)
            def _blk(b):
                sl = s + 16 * b

                @pl.when(sl < NSLAB)
                def _():
                    e0 = sl * KE
                    pltpu.sync_copy(msg_hbm.at[pl.ds(e0, KE)], msgb)
                    pltpu.sync_copy(dst_hbm.at[pl.ds(e0, KE)], dstb)
                    for k in range(NSB):
                        for t in range(SB // 16):
                            v = dstb[pl.ds(k * SB + t * 16, 16)]
                            loc = v - nbase
                            ok = (loc >= 0) & (loc < CH)
                            idxs[k][pl.ds(t * 16, 16)] = jnp.where(ok, loc, dvec)
                        pltpu.sync_copy(msgb.at[pl.ds(k * SB, SB)],
                                        acc.at[idxs[k]], add=True)

            plsc.subcore_barrier()
            pltpu.sync_copy(acc.at[pl.ds(s * ZPT, ZPT)],
                            out_hbm.at[pl.ds(nbase + s * ZPT, ZPT)])
            plsc.subcore_barrier()

    return pl.kernel(
        body,
        out_type=jax.ShapeDtypeStruct((NPAD, HP), jnp.float32),
        mesh=mesh,
        scratch_types=[
            pltpu.VMEM_SHARED((ACC, HP), jnp.float32),
            pltpu.VMEM((KE, HP), jnp.float32),
            pltpu.VMEM((KE,), jnp.int32),
        ] + [pltpu.VMEM((SB,), jnp.int32)] * NSB,
        compiler_params=pltpu.CompilerParams(use_tc_tiling_on_sc=False),
        interpret=interpret,
    )


def _make_gather(interpret=False):
    NB = -(-NSLAB // 32)
    mesh = plsc.VectorSubcoreMesh(core_axis_name="c", subcore_axis_name="s")

    def body(ns_hbm, src_hbm, out_hbm, rowsb, srcb, sem):
        c = lax.axis_index("c")
        s = lax.axis_index("s")
        w = s * 2 + c

        @pl.loop(0, NB)
        def _blk(b):
            sl = w + 32 * b

            @pl.when(sl < NSLAB)
            def _():
                e0 = sl * KE
                pltpu.sync_copy(src_hbm.at[pl.ds(e0, KE)], srcb)
                cps = [pltpu.async_copy(ns_hbm.at[srcb.at[pl.ds(j * SB, SB)]],
                                        rowsb.at[pl.ds(j * SB, SB)], sem)
                       for j in range(NSB)]
                for cp in cps:
                    cp.wait()
                pltpu.sync_copy(rowsb, out_hbm.at[pl.ds(e0, KE)])

    return pl.kernel(
        body,
        out_type=jax.ShapeDtypeStruct((E, HP), jnp.float32),
        mesh=mesh,
        scratch_types=[
            pltpu.VMEM((KE, HP), jnp.float32),
            pltpu.VMEM((KE,), jnp.int32),
            pltpu.SemaphoreType.DMA,
        ],
        compiler_params=pltpu.CompilerParams(use_tc_tiling_on_sc=False),
        interpret=interpret,
    )


def _tc_init(fb_p, wi2):
    def body(fb_ref, w_ref, bin_ref, msg_ref):
        t = jnp.dot(fb_ref[...], w_ref[...], preferred_element_type=jnp.float32)
        bin_ref[...] = t
        msg_ref[...] = jnp.maximum(t, 0.0)

    return pl.pallas_call(
        body,
        grid=((E // 2) // BE,),
        in_specs=[
            pl.BlockSpec((BE, 2 * H), lambda i: (i, 0)),
            pl.BlockSpec((2 * H, 2 * HP), lambda i: (0, 0)),
        ],
        out_specs=[pl.BlockSpec((BE, 2 * HP), lambda i: (i, 0))] * 2,
        out_shape=[jax.ShapeDtypeStruct((E // 2, 2 * HP), jnp.float32)] * 2,
    )(fb_p, wi2)


def _tc_round(nei_p, msg_p, bin_p, wh2, wh2s):
    def body(nei_ref, msg_ref, bin_ref, w_ref, ws_ref, out_ref):
        t = jnp.dot(nei_ref[...], w_ref[...], preferred_element_type=jnp.float32)
        t = t - jnp.dot(msg_ref[...], ws_ref[...],
                        preferred_element_type=jnp.float32)
        out_ref[...] = jnp.maximum(bin_ref[...] + t, 0.0)

    return pl.pallas_call(
        body,
        grid=((E // 2) // BE,),
        in_specs=[
            pl.BlockSpec((BE, 2 * HP), lambda i: (i, 0)),
            pl.BlockSpec((BE, 2 * HP), lambda i: (i, 0)),
            pl.BlockSpec((BE, 2 * HP), lambda i: (i, 0)),
            pl.BlockSpec((2 * HP, 2 * HP), lambda i: (0, 0)),
            pl.BlockSpec((2 * HP, 2 * HP), lambda i: (0, 0)),
        ],
        out_specs=pl.BlockSpec((BE, 2 * HP), lambda i: (i, 0)),
        out_shape=jax.ShapeDtypeStruct((E // 2, 2 * HP), jnp.float32),
    )(nei_p, msg_p, bin_p, wh2, wh2s)


def _tc_readout(fatoms, a_msg, mol3, wo1e, wo2e, boe):
    nblk = N // BN

    def body(fa_ref, am_ref, mol_ref, w1_ref, w2_ref, bo_ref, out_ref, acc):
        i = pl.program_id(0)
        h = jnp.dot(fa_ref[...], w1_ref[...], preferred_element_type=jnp.float32)
        h = h + jnp.dot(am_ref[...], w2_ref[...],
                        preferred_element_type=jnp.float32)
        h = jnp.maximum(h + bo_ref[...], 0.0)
        mol = mol_ref[0]
        oh = (mol == lax.broadcasted_iota(jnp.int32, (B, BN), 0)
              ).astype(jnp.float32)
        contrib = jnp.dot(oh, h, preferred_element_type=jnp.float32)

        @pl.when(i == 0)
        def _():
            acc[...] = jnp.zeros_like(acc)

        acc[...] += contrib

        @pl.when(i == nblk - 1)
        def _():
            a = acc[...]
            out_ref[...] = a[:, :H] / jnp.maximum(a[:, H:H + 1], 1.0)

    return pl.pallas_call(
        body,
        grid=(nblk,),
        in_specs=[
            pl.BlockSpec((BN, AF), lambda i: (i, 0)),
            pl.BlockSpec((BN, HP), lambda i: (i, 0)),
            pl.BlockSpec((1, 1, BN), lambda i: (i, 0, 0)),
            pl.BlockSpec((AF, H + 1), lambda i: (0, 0)),
            pl.BlockSpec((HP, H + 1), lambda i: (0, 0)),
            pl.BlockSpec((1, H + 1), lambda i: (0, 0)),
        ],
        out_specs=pl.BlockSpec((B, H), lambda i: (0, 0)),
        out_shape=jax.ShapeDtypeStruct((B, H), jnp.float32),
        scratch_shapes=[pltpu.VMEM((B, H + 1), jnp.float32)],
    )(fatoms, a_msg, mol3, wo1e, wo2e, boe)


def _block_diag_pad(w, rows):
    out = jnp.zeros((rows, 2 * HP), jnp.float32)
    out = out.at[0:H, 0:H].set(w)
    out = out.at[rows // 2:rows // 2 + H, HP:HP + H].set(w)
    return out


def _anti_block_diag_pad(w, rows):
    out = jnp.zeros((rows, 2 * HP), jnp.float32)
    out = out.at[0:H, HP:HP + H].set(w)
    out = out.at[rows // 2:rows // 2 + H, 0:H].set(w)
    return out


def kernel(fatoms, fbonds, edge_index, mol_ids, W_i, W_h, W_o, b_o):
    src1 = edge_index[0]
    dst1 = edge_index[1]
    fb_p = fbonds.reshape(E // 2, 2 * H)
    zer = jnp.zeros((ACC, HP), jnp.float32)

    wi2 = _block_diag_pad(W_i, 2 * H)
    wh2 = _block_diag_pad(W_h, 2 * HP)
    wh2s = _anti_block_diag_pad(W_h, 2 * HP)
    zcol = jnp.zeros((W_o.shape[0], 1), jnp.float32)
    wo_e = jnp.concatenate([W_o, zcol], 1)
    wo1e = wo_e[:AF]
    wo2e = jnp.zeros((HP, H + 1), jnp.float32).at[0:H].set(wo_e[AF:])
    boe = jnp.concatenate([b_o, jnp.ones((1,), jnp.float32)])[None, :]

    segsum = _make_segsum()
    gather = _make_gather()

    bin_p, msg_p = _tc_init(fb_p, wi2)
    for _ in range(DEPTH - 1):
        ns = segsum(msg_p.reshape(E, HP), dst1, zer)
        nei = gather(ns, src1)
        msg_p = _tc_round(nei.reshape(E // 2, 2 * HP), msg_p, bin_p, wh2, wh2s)

    a_msg = segsum(msg_p.reshape(E, HP), dst1, zer)
    mol3 = mol_ids.reshape(N // BN, 1, BN)
    return _tc_readout(fatoms, a_msg, mol3, wo1e, wo2e, boe)


# speedup vs baseline: 3.9709x; 1.3869x over previous
import functools

import jax
import jax.numpy as jnp
from jax import lax
from jax.experimental import pallas as pl
from jax.experimental.pallas import tpu as pltpu
from jax.experimental.pallas import tpu_sc as plsc

N = 100000
E = 1600000
AF = 39
H = 50
B = 1024
DEPTH = 3

CH = 25600
NPAD = 4 * CH
ACC = CH + 16
KE = 320
SB = 80
NSB = KE // SB
NSLAB = E // KE
HP = 64
BE = 4000
BN = 1000


def _make_segsum(interpret=False):
    KS = 160
    NSLAB_S = E // KS
    NBS = NSLAB_S // 16
    NPAIR = (NBS - 1) // 2
    NSBS = KS // SB
    ZPT = CH // 16
    mesh = plsc.VectorSubcoreMesh(core_axis_name="c", subcore_axis_name="s")

    def body(msg_hbm, dst_hbm, zer_hbm, out_hbm, acc,
             m0, m1, d0, d1, x0, x1, sem0, sem1):
        c = lax.axis_index("c")
        s = lax.axis_index("s")
        dvec = CH + lax.broadcasted_iota(jnp.int32, (16,), 0)
        bufs = ((m0, d0, x0, sem0), (m1, d1, x1, sem1))

        def start(j, p):
            mb, db, _, sem = bufs[p]
            e0 = (s + 16 * j) * KS
            pltpu.async_copy(msg_hbm.at[pl.ds(e0, KS)], mb, sem)
            pltpu.async_copy(dst_hbm.at[pl.ds(e0, KS)], db, sem)

        def wait(p):
            mb, db, _, sem = bufs[p]
            pltpu.make_async_copy(msg_hbm.at[pl.ds(0, KS)], mb, sem).wait()
            pltpu.make_async_copy(dst_hbm.at[pl.ds(0, KS)], db, sem).wait()

        def process(p, nbase):
            mb, db, xk, _ = bufs[p]
            for k in range(NSBS):
                for t in range(SB // 16):
                    v = db[pl.ds(k * SB + t * 16, 16)]
                    loc = v - nbase
                    ok = (loc >= 0) & (loc < CH)
                    xk[pl.ds(t * 16, 16)] = jnp.where(ok, loc, dvec)
                pltpu.sync_copy(mb.at[pl.ds(k * SB, SB)],
                                acc.at[xk], add=True)

        for half in range(2):
            ch = c + 2 * half
            nbase = ch * CH
            pltpu.sync_copy(zer_hbm.at[pl.ds(s * ZPT, ZPT)],
                            acc.at[pl.ds(s * ZPT, ZPT)])

            @pl.when(s == 0)
            def _():
                pltpu.sync_copy(zer_hbm.at[pl.ds(0, 16)], acc.at[pl.ds(CH, 16)])

            plsc.subcore_barrier()
            start(0, 0)

            @pl.loop(0, NPAIR)
            def _blk(i):
                j0 = 2 * i
                start(j0 + 1, 1)
                wait(0)
                process(0, nbase)
                start(j0 + 2, 0)
                wait(1)
                process(1, nbase)

            wait(0)
            process(0, nbase)
            plsc.subcore_barrier()
            pltpu.sync_copy(acc.at[pl.ds(s * ZPT, ZPT)],
                            out_hbm.at[pl.ds(nbase + s * ZPT, ZPT)])
            plsc.subcore_barrier()

    return pl.kernel(
        body,
        out_type=jax.ShapeDtypeStruct((NPAD, HP), jnp.float32),
        mesh=mesh,
        scratch_types=[
            pltpu.VMEM_SHARED((ACC, HP), jnp.float32),
            pltpu.VMEM((KS, HP), jnp.float32),
            pltpu.VMEM((KS, HP), jnp.float32),
            pltpu.VMEM((KS,), jnp.int32),
            pltpu.VMEM((KS,), jnp.int32),
            pltpu.VMEM((SB,), jnp.int32),
            pltpu.VMEM((SB,), jnp.int32),
            pltpu.SemaphoreType.DMA,
            pltpu.SemaphoreType.DMA,
        ],
        compiler_params=pltpu.CompilerParams(use_tc_tiling_on_sc=False),
        interpret=interpret,
    )


def _make_gather(interpret=False):
    NB = -(-NSLAB // 32)
    mesh = plsc.VectorSubcoreMesh(core_axis_name="c", subcore_axis_name="s")

    def body(ns_hbm, src_hbm, out_hbm, rowsb, srcb, sem):
        c = lax.axis_index("c")
        s = lax.axis_index("s")
        w = s * 2 + c

        @pl.loop(0, NB)
        def _blk(b):
            sl = w + 32 * b

            @pl.when(sl < NSLAB)
            def _():
                e0 = sl * KE
                pltpu.sync_copy(src_hbm.at[pl.ds(e0, KE)], srcb)
                cps = [pltpu.async_copy(ns_hbm.at[srcb.at[pl.ds(j * SB, SB)]],
                                        rowsb.at[pl.ds(j * SB, SB)], sem)
                       for j in range(NSB)]
                for cp in cps:
                    cp.wait()
                pltpu.sync_copy(rowsb, out_hbm.at[pl.ds(e0, KE)])

    return pl.kernel(
        body,
        out_type=jax.ShapeDtypeStruct((E, HP), jnp.float32),
        mesh=mesh,
        scratch_types=[
            pltpu.VMEM((KE, HP), jnp.float32),
            pltpu.VMEM((KE,), jnp.int32),
            pltpu.SemaphoreType.DMA,
        ],
        compiler_params=pltpu.CompilerParams(use_tc_tiling_on_sc=False),
        interpret=interpret,
    )


def _tc_init(fb_p, wi2):
    def body(fb_ref, w_ref, bin_ref, msg_ref):
        t = jnp.dot(fb_ref[...], w_ref[...], preferred_element_type=jnp.float32)
        bin_ref[...] = t
        msg_ref[...] = jnp.maximum(t, 0.0)

    return pl.pallas_call(
        body,
        grid=((E // 2) // BE,),
        in_specs=[
            pl.BlockSpec((BE, 2 * H), lambda i: (i, 0)),
            pl.BlockSpec((2 * H, 2 * HP), lambda i: (0, 0)),
        ],
        out_specs=[pl.BlockSpec((BE, 2 * HP), lambda i: (i, 0))] * 2,
        out_shape=[jax.ShapeDtypeStruct((E // 2, 2 * HP), jnp.float32)] * 2,
    )(fb_p, wi2)


def _tc_round(nei_p, msg_p, bin_p, wh2, wh2s):
    def body(nei_ref, msg_ref, bin_ref, w_ref, ws_ref, out_ref):
        t = jnp.dot(nei_ref[...], w_ref[...], preferred_element_type=jnp.float32)
        t = t - jnp.dot(msg_ref[...], ws_ref[...],
                        preferred_element_type=jnp.float32)
        out_ref[...] = jnp.maximum(bin_ref[...] + t, 0.0)

    return pl.pallas_call(
        body,
        grid=((E // 2) // BE,),
        in_specs=[
            pl.BlockSpec((BE, 2 * HP), lambda i: (i, 0)),
            pl.BlockSpec((BE, 2 * HP), lambda i: (i, 0)),
            pl.BlockSpec((BE, 2 * HP), lambda i: (i, 0)),
            pl.BlockSpec((2 * HP, 2 * HP), lambda i: (0, 0)),
            pl.BlockSpec((2 * HP, 2 * HP), lambda i: (0, 0)),
        ],
        out_specs=pl.BlockSpec((BE, 2 * HP), lambda i: (i, 0)),
        out_shape=jax.ShapeDtypeStruct((E // 2, 2 * HP), jnp.float32),
    )(nei_p, msg_p, bin_p, wh2, wh2s)


def _tc_readout(fatoms, a_msg, mol3, wo1e, wo2e, boe):
    nblk = N // BN

    def body(fa_ref, am_ref, mol_ref, w1_ref, w2_ref, bo_ref, out_ref, acc):
        i = pl.program_id(0)
        h = jnp.dot(fa_ref[...], w1_ref[...], preferred_element_type=jnp.float32)
        h = h + jnp.dot(am_ref[...], w2_ref[...],
                        preferred_element_type=jnp.float32)
        h = jnp.maximum(h + bo_ref[...], 0.0)
        mol = mol_ref[0]
        oh = (mol == lax.broadcasted_iota(jnp.int32, (B, BN), 0)
              ).astype(jnp.float32)
        contrib = jnp.dot(oh, h, preferred_element_type=jnp.float32)

        @pl.when(i == 0)
        def _():
            acc[...] = jnp.zeros_like(acc)

        acc[...] += contrib

        @pl.when(i == nblk - 1)
        def _():
            a = acc[...]
            out_ref[...] = a[:, :H] / jnp.maximum(a[:, H:H + 1], 1.0)

    return pl.pallas_call(
        body,
        grid=(nblk,),
        in_specs=[
            pl.BlockSpec((BN, AF), lambda i: (i, 0)),
            pl.BlockSpec((BN, HP), lambda i: (i, 0)),
            pl.BlockSpec((1, 1, BN), lambda i: (i, 0, 0)),
            pl.BlockSpec((AF, H + 1), lambda i: (0, 0)),
            pl.BlockSpec((HP, H + 1), lambda i: (0, 0)),
            pl.BlockSpec((1, H + 1), lambda i: (0, 0)),
        ],
        out_specs=pl.BlockSpec((B, H), lambda i: (0, 0)),
        out_shape=jax.ShapeDtypeStruct((B, H), jnp.float32),
        scratch_shapes=[pltpu.VMEM((B, H + 1), jnp.float32)],
    )(fatoms, a_msg, mol3, wo1e, wo2e, boe)


def _block_diag_pad(w, rows):
    out = jnp.zeros((rows, 2 * HP), jnp.float32)
    out = out.at[0:H, 0:H].set(w)
    out = out.at[rows // 2:rows // 2 + H, HP:HP + H].set(w)
    return out


def _anti_block_diag_pad(w, rows):
    out = jnp.zeros((rows, 2 * HP), jnp.float32)
    out = out.at[0:H, HP:HP + H].set(w)
    out = out.at[rows // 2:rows // 2 + H, 0:H].set(w)
    return out


def kernel(fatoms, fbonds, edge_index, mol_ids, W_i, W_h, W_o, b_o):
    src1 = edge_index[0]
    dst1 = edge_index[1]
    fb_p = fbonds.reshape(E // 2, 2 * H)
    zer = jnp.zeros((ACC, HP), jnp.float32)

    wi2 = _block_diag_pad(W_i, 2 * H)
    wh2 = _block_diag_pad(W_h, 2 * HP)
    wh2s = _anti_block_diag_pad(W_h, 2 * HP)
    zcol = jnp.zeros((W_o.shape[0], 1), jnp.float32)
    wo_e = jnp.concatenate([W_o, zcol], 1)
    wo1e = wo_e[:AF]
    wo2e = jnp.zeros((HP, H + 1), jnp.float32).at[0:H].set(wo_e[AF:])
    boe = jnp.concatenate([b_o, jnp.ones((1,), jnp.float32)])[None, :]

    segsum = _make_segsum()
    gather = _make_gather()

    bin_p, msg_p = _tc_init(fb_p, wi2)
    for _ in range(DEPTH - 1):
        ns = segsum(msg_p.reshape(E, HP), dst1, zer)
        nei = gather(ns, src1)
        msg_p = _tc_round(nei.reshape(E // 2, 2 * HP), msg_p, bin_p, wh2, wh2s)

    a_msg = segsum(msg_p.reshape(E, HP), dst1, zer)
    mol3 = mol_ids.reshape(N // BN, 1, BN)
    return _tc_readout(fatoms, a_msg, mol3, wo1e, wo2e, boe)


# speedup vs baseline: 4.2478x; 1.0697x over previous
import functools

import jax
import jax.numpy as jnp
from jax import lax
from jax.experimental import pallas as pl
from jax.experimental.pallas import tpu as pltpu
from jax.experimental.pallas import tpu_sc as plsc

N = 100000
E = 1600000
AF = 39
H = 50
B = 1024
DEPTH = 3

CH = 25600
NPAD = 4 * CH
ACC = CH + 16
KE = 320
SB = 80
NSB = KE // SB
NSLAB = E // KE
HP = 64
BE = 4000
BN = 1000


def _make_segsum(interpret=False):
    KS = 160
    NSLAB_S = E // KS
    NBS = NSLAB_S // 16
    NPAIR = (NBS - 1) // 2
    NSBS = KS // SB
    ZPT = CH // 16
    mesh = plsc.VectorSubcoreMesh(core_axis_name="c", subcore_axis_name="s")

    def body(msg_hbm, dst_hbm, zer_hbm, out_hbm, acc,
             m0, m1, d0, d1, x0, x1, sem0, sem1):
        c = lax.axis_index("c")
        s = lax.axis_index("s")
        dvec = CH + lax.broadcasted_iota(jnp.int32, (16,), 0)
        bufs = ((m0, d0, x0, sem0), (m1, d1, x1, sem1))

        def start(j, p):
            mb, db, _, sem = bufs[p]
            e0 = (s + 16 * j) * KS
            pltpu.async_copy(msg_hbm.at[pl.ds(e0, KS)], mb, sem)
            pltpu.async_copy(dst_hbm.at[pl.ds(e0, KS)], db, sem)

        def wait(p):
            mb, db, _, sem = bufs[p]
            pltpu.make_async_copy(msg_hbm.at[pl.ds(0, KS)], mb, sem).wait()
            pltpu.make_async_copy(dst_hbm.at[pl.ds(0, KS)], db, sem).wait()

        def process(p, nbase):
            mb, db, xk, _ = bufs[p]
            for k in range(NSBS):
                for t in range(SB // 16):
                    v = db[pl.ds(k * SB + t * 16, 16)]
                    loc = v - nbase
                    ok = (loc >= 0) & (loc < CH)
                    xk[pl.ds(t * 16, 16)] = jnp.where(ok, loc, dvec)
                pltpu.sync_copy(mb.at[pl.ds(k * SB, SB)],
                                acc.at[xk], add=True)

        for half in range(2):
            ch = c + 2 * half
            nbase = ch * CH
            pltpu.sync_copy(zer_hbm.at[pl.ds(s * ZPT, ZPT)],
                            acc.at[pl.ds(s * ZPT, ZPT)])

            @pl.when(s == 0)
            def _():
                pltpu.sync_copy(zer_hbm.at[pl.ds(0, 16)], acc.at[pl.ds(CH, 16)])

            plsc.subcore_barrier()
            start(0, 0)

            @pl.loop(0, NPAIR)
            def _blk(i):
                j0 = 2 * i
                start(j0 + 1, 1)
                wait(0)
                process(0, nbase)
                start(j0 + 2, 0)
                wait(1)
                process(1, nbase)

            wait(0)
            process(0, nbase)
            plsc.subcore_barrier()
            pltpu.sync_copy(acc.at[pl.ds(s * ZPT, ZPT)],
                            out_hbm.at[pl.ds(nbase + s * ZPT, ZPT)])
            plsc.subcore_barrier()

    return pl.kernel(
        body,
        out_type=jax.ShapeDtypeStruct((NPAD, HP), jnp.float32),
        mesh=mesh,
        scratch_types=[
            pltpu.VMEM_SHARED((ACC, HP), jnp.float32),
            pltpu.VMEM((KS, HP), jnp.float32),
            pltpu.VMEM((KS, HP), jnp.float32),
            pltpu.VMEM((KS,), jnp.int32),
            pltpu.VMEM((KS,), jnp.int32),
            pltpu.VMEM((SB,), jnp.int32),
            pltpu.VMEM((SB,), jnp.int32),
            pltpu.SemaphoreType.DMA,
            pltpu.SemaphoreType.DMA,
        ],
        compiler_params=pltpu.CompilerParams(use_tc_tiling_on_sc=False),
        interpret=interpret,
    )


def _make_gather(interpret=False):
    KG = 400
    EPT = E // 32
    NBG = EPT // KG
    NPG = (NBG - 1) // 2
    NSG = KG // SB
    mesh = plsc.VectorSubcoreMesh(core_axis_name="c", subcore_axis_name="s")

    def body(ns_hbm, src_hbm, out_hbm, r0, r1, s0, s1,
             seml0, seml1, semg0, semg1):
        c = lax.axis_index("c")
        s = lax.axis_index("s")
        base = (s * 2 + c) * EPT
        bufs = ((r0, s0, seml0, semg0), (r1, s1, seml1, semg1))

        def start_l(j, p):
            _, sb, seml, _ = bufs[p]
            pltpu.async_copy(src_hbm.at[pl.ds(base + j * KG, KG)], sb, seml)

        def wait_l(p):
            _, sb, seml, _ = bufs[p]
            pltpu.make_async_copy(src_hbm.at[pl.ds(0, KG)], sb, seml).wait()

        def fire_g(p):
            rb, sb, _, semg = bufs[p]
            for k in range(NSG):
                pltpu.async_copy(ns_hbm.at[sb.at[pl.ds(k * SB, SB)]],
                                 rb.at[pl.ds(k * SB, SB)], semg)

        def drain_store(j, p):
            rb, sb, _, semg = bufs[p]
            for k in range(NSG):
                pltpu.make_async_copy(ns_hbm.at[sb.at[pl.ds(k * SB, SB)]],
                                      rb.at[pl.ds(k * SB, SB)], semg).wait()
            pltpu.sync_copy(rb, out_hbm.at[pl.ds(base + j * KG, KG)])

        start_l(0, 0)
        wait_l(0)
        fire_g(0)
        start_l(1, 1)

        @pl.loop(0, NPG)
        def _blk(i):
            j0 = 2 * i
            wait_l(1)
            fire_g(1)
            drain_store(j0, 0)
            start_l(j0 + 2, 0)
            wait_l(0)
            fire_g(0)
            drain_store(j0 + 1, 1)

            @pl.when(j0 + 3 < NBG)
            def _():
                start_l(j0 + 3, 1)

        drain_store(NBG - 1, 0)

    return pl.kernel(
        body,
        out_type=jax.ShapeDtypeStruct((E, HP), jnp.float32),
        mesh=mesh,
        scratch_types=[
            pltpu.VMEM((KG, HP), jnp.float32),
            pltpu.VMEM((KG, HP), jnp.float32),
            pltpu.VMEM((KG,), jnp.int32),
            pltpu.VMEM((KG,), jnp.int32),
            pltpu.SemaphoreType.DMA,
            pltpu.SemaphoreType.DMA,
            pltpu.SemaphoreType.DMA,
            pltpu.SemaphoreType.DMA,
        ],
        compiler_params=pltpu.CompilerParams(use_tc_tiling_on_sc=False),
        interpret=interpret,
    )


def _tc_init(fb_p, wi2):
    def body(fb_ref, w_ref, bin_ref, msg_ref):
        t = jnp.dot(fb_ref[...], w_ref[...], preferred_element_type=jnp.float32)
        bin_ref[...] = t
        msg_ref[...] = jnp.maximum(t, 0.0)

    return pl.pallas_call(
        body,
        grid=((E // 2) // BE,),
        in_specs=[
            pl.BlockSpec((BE, 2 * H), lambda i: (i, 0)),
            pl.BlockSpec((2 * H, 2 * HP), lambda i: (0, 0)),
        ],
        out_specs=[pl.BlockSpec((BE, 2 * HP), lambda i: (i, 0))] * 2,
        out_shape=[jax.ShapeDtypeStruct((E // 2, 2 * HP), jnp.float32)] * 2,
    )(fb_p, wi2)


def _tc_round1(nei_p, bin_p, wh2, wh2s):
    def body(nei_ref, bin_ref, w_ref, ws_ref, out_ref):
        bv = bin_ref[...]
        t = jnp.dot(nei_ref[...], w_ref[...], preferred_element_type=jnp.float32)
        t = t - jnp.dot(jnp.maximum(bv, 0.0), ws_ref[...],
                        preferred_element_type=jnp.float32)
        out_ref[...] = jnp.maximum(bv + t, 0.0)

    return pl.pallas_call(
        body,
        grid=((E // 2) // BE,),
        in_specs=[
            pl.BlockSpec((BE, 2 * HP), lambda i: (i, 0)),
            pl.BlockSpec((BE, 2 * HP), lambda i: (i, 0)),
            pl.BlockSpec((2 * HP, 2 * HP), lambda i: (0, 0)),
            pl.BlockSpec((2 * HP, 2 * HP), lambda i: (0, 0)),
        ],
        out_specs=pl.BlockSpec((BE, 2 * HP), lambda i: (i, 0)),
        out_shape=jax.ShapeDtypeStruct((E // 2, 2 * HP), jnp.float32),
    )(nei_p, bin_p, wh2, wh2s)


def _tc_round(nei_p, msg_p, bin_p, wh2, wh2s):
    def body(nei_ref, msg_ref, bin_ref, w_ref, ws_ref, out_ref):
        t = jnp.dot(nei_ref[...], w_ref[...], preferred_element_type=jnp.float32)
        t = t - jnp.dot(msg_ref[...], ws_ref[...],
                        preferred_element_type=jnp.float32)
        out_ref[...] = jnp.maximum(bin_ref[...] + t, 0.0)

    return pl.pallas_call(
        body,
        grid=((E // 2) // BE,),
        in_specs=[
            pl.BlockSpec((BE, 2 * HP), lambda i: (i, 0)),
            pl.BlockSpec((BE, 2 * HP), lambda i: (i, 0)),
            pl.BlockSpec((BE, 2 * HP), lambda i: (i, 0)),
            pl.BlockSpec((2 * HP, 2 * HP), lambda i: (0, 0)),
            pl.BlockSpec((2 * HP, 2 * HP), lambda i: (0, 0)),
        ],
        out_specs=pl.BlockSpec((BE, 2 * HP), lambda i: (i, 0)),
        out_shape=jax.ShapeDtypeStruct((E // 2, 2 * HP), jnp.float32),
    )(nei_p, msg_p, bin_p, wh2, wh2s)


def _tc_readout(fatoms, a_msg, mol3, wo1e, wo2e, boe):
    nblk = N // BN

    def body(fa_ref, am_ref, mol_ref, w1_ref, w2_ref, bo_ref, out_ref, acc):
        i = pl.program_id(0)
        h = jnp.dot(fa_ref[...], w1_ref[...], preferred_element_type=jnp.float32)
        h = h + jnp.dot(am_ref[...], w2_ref[...],
                        preferred_element_type=jnp.float32)
        h = jnp.maximum(h + bo_ref[...], 0.0)
        mol = mol_ref[0]
        oh = (mol == lax.broadcasted_iota(jnp.int32, (B, BN), 0)
              ).astype(jnp.float32)
        contrib = jnp.dot(oh, h, preferred_element_type=jnp.float32)

        @pl.when(i == 0)
        def _():
            acc[...] = jnp.zeros_like(acc)

        acc[...] += contrib

        @pl.when(i == nblk - 1)
        def _():
            a = acc[...]
            out_ref[...] = a[:, :H] / jnp.maximum(a[:, H:H + 1], 1.0)

    return pl.pallas_call(
        body,
        grid=(nblk,),
        in_specs=[
            pl.BlockSpec((BN, AF), lambda i: (i, 0)),
            pl.BlockSpec((BN, HP), lambda i: (i, 0)),
            pl.BlockSpec((1, 1, BN), lambda i: (i, 0, 0)),
            pl.BlockSpec((AF, H + 1), lambda i: (0, 0)),
            pl.BlockSpec((HP, H + 1), lambda i: (0, 0)),
            pl.BlockSpec((1, H + 1), lambda i: (0, 0)),
        ],
        out_specs=pl.BlockSpec((B, H), lambda i: (0, 0)),
        out_shape=jax.ShapeDtypeStruct((B, H), jnp.float32),
        scratch_shapes=[pltpu.VMEM((B, H + 1), jnp.float32)],
    )(fatoms, a_msg, mol3, wo1e, wo2e, boe)


def _block_diag_pad(w, rows):
    out = jnp.zeros((rows, 2 * HP), jnp.float32)
    out = out.at[0:H, 0:H].set(w)
    out = out.at[rows // 2:rows // 2 + H, HP:HP + H].set(w)
    return out


def _anti_block_diag_pad(w, rows):
    out = jnp.zeros((rows, 2 * HP), jnp.float32)
    out = out.at[0:H, HP:HP + H].set(w)
    out = out.at[rows // 2:rows // 2 + H, 0:H].set(w)
    return out


def kernel(fatoms, fbonds, edge_index, mol_ids, W_i, W_h, W_o, b_o):
    src1 = edge_index[0]
    dst1 = edge_index[1]
    fb_p = fbonds.reshape(E // 2, 2 * H)
    zer = jnp.zeros((ACC, HP), jnp.float32)

    wi2 = _block_diag_pad(W_i, 2 * H)
    wh2 = _block_diag_pad(W_h, 2 * HP)
    wh2s = _anti_block_diag_pad(W_h, 2 * HP)
    zcol = jnp.zeros((W_o.shape[0], 1), jnp.float32)
    wo_e = jnp.concatenate([W_o, zcol], 1)
    wo1e = wo_e[:AF]
    wo2e = jnp.zeros((HP, H + 1), jnp.float32).at[0:H].set(wo_e[AF:])
    boe = jnp.concatenate([b_o, jnp.ones((1,), jnp.float32)])[None, :]

    segsum = _make_segsum()
    gather = _make_gather()

    bin_p, msg_p = _tc_init(fb_p, wi2)
    for r in range(DEPTH - 1):
        ns = segsum(msg_p.reshape(E, HP), dst1, zer)
        nei = gather(ns, src1)
        nei_p = nei.reshape(E // 2, 2 * HP)
        if r == 0:
            msg_p = _tc_round1(nei_p, bin_p, wh2, wh2s)
        else:
            msg_p = _tc_round(nei_p, msg_p, bin_p, wh2, wh2s)

    a_msg = segsum(msg_p.reshape(E, HP), dst1, zer)
    mol3 = mol_ids.reshape(N // BN, 1, BN)
    return _tc_readout(fatoms, a_msg, mol3, wo1e, wo2e, boe)
